# Initial kernel scaffold; baseline (speedup 1.0000x reference)
import jax
import jax.numpy as jnp
from jax.experimental import pallas as pl


def kernel(x, rows, cols, vals):
    raise NotImplementedError("write your pallas kernel here")


# speedup vs baseline: 10.6657x; 10.6657x over previous
import functools

import jax
import jax.numpy as jnp
from jax import lax
from jax.experimental import pallas as pl
from jax.experimental.pallas import tpu as pltpu
from jax.experimental.pallas import tpu_sc as plsc

OUT_LAT, OUT_LON = 91, 180
N_OUT = OUT_LAT * OUT_LON
K = 6
BC = 128
L = 16
NC, NS = 2, 16
NW = NC * NS
N_OUT_PAD = 16384
ROWS_PER_W = N_OUT_PAD // NW
ROWS_PER_CHUNK = 16
CHUNKS = ROWS_PER_W // ROWS_PER_CHUNK
E_PER_CHUNK = ROWS_PER_CHUNK * K
E_PER_W = ROWS_PER_W * K
VREGS_PER_ROW = BC // L


def _bcast_lane(vec, lane):
    idx = jnp.full((L, 1), lane, dtype=jnp.int32)
    dn = lax.GatherDimensionNumbers(
        offset_dims=(), collapsed_slice_dims=(0,), start_index_map=(0,)
    )
    return lax.gather(
        vec, idx, dn, slice_sizes=(1,),
        mode=lax.GatherScatterMode.PROMISE_IN_BOUNDS,
    )


_MESH = plsc.VectorSubcoreMesh(core_axis_name="c", subcore_axis_name="s")


@functools.partial(
    pl.kernel,
    mesh=_MESH,
    out_type=jax.ShapeDtypeStruct((N_OUT_PAD, BC), jnp.float32),
    scratch_types=[
        pltpu.VMEM((CHUNKS, E_PER_CHUNK), jnp.int32),
        pltpu.VMEM((E_PER_W,), jnp.float32),
        pltpu.VMEM((E_PER_CHUNK, BC), jnp.float32),
        pltpu.VMEM((ROWS_PER_W, BC), jnp.float32),
        pltpu.SemaphoreType.DMA,
    ],
)
def _regrid(xT, colsr, valsr, out, idx_v, vals_v, g, outb, sem):
    wid = lax.axis_index("s") * NC + lax.axis_index("c")
    pltpu.sync_copy(colsr.at[pl.ds(wid * CHUNKS, CHUNKS)], idx_v)
    pltpu.sync_copy(valsr.at[pl.ds(wid * E_PER_W, E_PER_W)], vals_v)

    def chunk_body(t, carry):
        pltpu.async_copy(xT.at[idx_v.at[t]], g, sem).wait()
        for grp in range(ROWS_PER_CHUNK // 8):
            vv = [
                vals_v[pl.ds(t * E_PER_CHUNK + grp * 48 + L * v, L)]
                for v in range(3)
            ]
            for r in range(8):
                acc = [None] * VREGS_PER_ROW
                for k in range(K):
                    lane = K * r + k
                    wgt = _bcast_lane(vv[lane // L], lane % L)
                    e = grp * 48 + lane
                    for j in range(VREGS_PER_ROW):
                        gv = g[e, pl.ds(L * j, L)]
                        acc[j] = wgt * gv if acc[j] is None else acc[j] + wgt * gv
                row = t * ROWS_PER_CHUNK + grp * 8 + r
                for j in range(VREGS_PER_ROW):
                    outb[row, pl.ds(L * j, L)] = acc[j]
        return carry

    lax.fori_loop(0, CHUNKS, chunk_body, 0)
    pltpu.sync_copy(outb, out.at[pl.ds(wid * ROWS_PER_W, ROWS_PER_W)])


def kernel(x, rows, cols, vals):
    b, c, h, w = x.shape
    xT = x.reshape(b * c, h * w).T
    pad = N_OUT_PAD * K - cols.shape[0]
    cols_p = jnp.concatenate([cols, jnp.zeros((pad,), cols.dtype)])
    vals_p = jnp.concatenate([vals, jnp.zeros((pad,), vals.dtype)])
    yT = _regrid(xT, cols_p.reshape(NW * CHUNKS, E_PER_CHUNK), vals_p)
    return yT[:N_OUT].T.reshape(b, c, OUT_LAT, OUT_LON)
